# baseline (speedup 1.0000x reference)
import functools
import math

import jax
import jax.numpy as jnp
from jax import lax
from jax.experimental import pallas as pl
from jax.experimental.pallas import tpu as pltpu
from jax.experimental.pallas import tpu_sc as plsc

_NUM_HEADS = 16
_NUM_SEGMENTS = 32
_NUM_BUCKETS = 32
_MAX_DISTANCE = 128
_Q = 2048
_K = 2048
_ABS_LEN = 2048
_SEG_LEN = _NUM_SEGMENTS * _NUM_SEGMENTS
_LUT_LEN = _ABS_LEN + _SEG_LEN
_TABLE_ROWS = _SEG_LEN + _NUM_BUCKETS

_NUM_WORKERS = 32
_ROWS_PER_WORKER = _Q // _NUM_WORKERS
_LANES = 16


def _lut_body(table_ref, out_ref):
    i = lax.broadcasted_iota(jnp.int32, (_TABLE_ROWS, _LUT_LEN), 1)
    b = lax.broadcasted_iota(jnp.int32, (_TABLE_ROWS, _LUT_LEN), 0)
    max_exact = _NUM_BUCKETS // 2
    safe_rp = jnp.maximum(i.astype(jnp.float32), 1.0)
    rp_if_large = max_exact + (
        jnp.log(safe_rp / max_exact)
        / math.log(_MAX_DISTANCE / max_exact)
        * (_NUM_BUCKETS - max_exact)
    ).astype(jnp.int32)
    rp_if_large = jnp.minimum(rp_if_large, _NUM_BUCKETS - 1)
    abs_bucket = jnp.where(i < max_exact, i, rp_if_large)
    idx = jnp.where(i < _ABS_LEN, abs_bucket, i - _ABS_LEN + _NUM_BUCKETS)
    onehot = (idx == b).astype(jnp.float32)
    out_ref[...] = lax.dot_general(
        table_ref[...],
        onehot,
        (((0,), (0,)), ((), ())),
        preferred_element_type=jnp.float32,
        precision=lax.Precision.HIGHEST,
    )


def _build_lut(table):
    return pl.pallas_call(
        _lut_body,
        out_shape=jax.ShapeDtypeStruct((_NUM_HEADS, _LUT_LEN), jnp.float32),
    )(table)


def _sc_body(
    lut_hbm, qseg_hbm, kseg_hbm, out_hbm, lut_v, ks_v, qs_v, row_a, row_b,
    sem_a, sem_b,
):
    c = lax.axis_index("c")
    s = lax.axis_index("s")
    w = s * 2 + c
    base_q = w * _ROWS_PER_WORKER

    pltpu.sync_copy(lut_hbm, lut_v)
    pltpu.sync_copy(kseg_hbm, ks_v)
    pltpu.sync_copy(qseg_hbm.at[pl.ds(base_q, _ROWS_PER_WORKER)], qs_v)

    lane_iota = lax.iota(jnp.int32, _LANES)

    def compute_row(qq, qi, buf):
        qsv = plsc.load_gather(qs_v, [jnp.broadcast_to(qi, (_LANES,))])
        segbase = _ABS_LEN + qsv * _NUM_SEGMENTS
        dbase = qq - lane_iota

        @plsc.parallel_loop(0, _K, _LANES, unroll=8)
        def k_body(k0):
            kv = ks_v[pl.ds(k0, _LANES)]
            dcl = jnp.clip(dbase - k0, 0, _ABS_LEN - 1)
            idx = jnp.where(qsv == kv, dcl, segbase + kv)
            for h in range(_NUM_HEADS):
                val = plsc.load_gather(
                    lut_v.at[pl.ds(h * _LUT_LEN, _LUT_LEN)], [idx]
                )
                buf[h, 0, pl.ds(k0, _LANES)] = val

    def drain(buf, sem):
        pltpu.make_async_copy(out_hbm.at[:, pl.ds(0, 1), :], buf, sem).wait()

    def q_body(qi, carry):
        q0 = base_q + 2 * qi

        @pl.when(qi > 0)
        def _():
            drain(row_a, sem_a)

        compute_row(q0, 2 * qi, row_a)
        pltpu.async_copy(row_a, out_hbm.at[:, pl.ds(q0, 1), :], sem_a)

        @pl.when(qi > 0)
        def _():
            drain(row_b, sem_b)

        compute_row(q0 + 1, 2 * qi + 1, row_b)
        pltpu.async_copy(row_b, out_hbm.at[:, pl.ds(q0 + 1, 1), :], sem_b)
        return carry

    lax.fori_loop(0, _ROWS_PER_WORKER // 2, q_body, 0)
    drain(row_a, sem_a)
    drain(row_b, sem_b)


@jax.jit
def _run(qseg, kseg, table):
    lut = _build_lut(table).reshape(-1)
    mesh = plsc.VectorSubcoreMesh(core_axis_name="c", subcore_axis_name="s")
    out = pl.kernel(
        _sc_body,
        out_type=jax.ShapeDtypeStruct((_NUM_HEADS, _Q, _K), jnp.float32),
        mesh=mesh,
        compiler_params=pltpu.CompilerParams(needs_layout_passes=False),
        scratch_types=[
            pltpu.VMEM((_NUM_HEADS * _LUT_LEN,), jnp.float32),
            pltpu.VMEM((_K,), jnp.int32),
            pltpu.VMEM((_ROWS_PER_WORKER,), jnp.int32),
            pltpu.VMEM((_NUM_HEADS, 1, _K), jnp.float32),
            pltpu.VMEM((_NUM_HEADS, 1, _K), jnp.float32),
            pltpu.SemaphoreType.DMA,
            pltpu.SemaphoreType.DMA,
        ],
    )(lut, qseg, kseg)
    return out


def kernel(key_pos, query_pos, key_segment, query_segment, relative_attention_bias):
    del key_pos, query_pos
    qseg = query_segment.reshape(_Q)
    kseg = key_segment.reshape(_K)
    out = _run(qseg, kseg, relative_attention_bias)
    return out.reshape(1, _NUM_HEADS, _Q, _K)


# speedup vs baseline: 1.0113x; 1.0113x over previous
import functools
import math

import jax
import jax.numpy as jnp
from jax import lax
from jax.experimental import pallas as pl
from jax.experimental.pallas import tpu as pltpu
from jax.experimental.pallas import tpu_sc as plsc

_NUM_HEADS = 16
_NUM_SEGMENTS = 32
_NUM_BUCKETS = 32
_MAX_DISTANCE = 128
_Q = 2048
_K = 2048
_ABS_LEN = 2048
_SEG_LEN = _NUM_SEGMENTS * _NUM_SEGMENTS
_LUT_LEN = _ABS_LEN + _SEG_LEN
_TABLE_ROWS = _SEG_LEN + _NUM_BUCKETS

_NUM_WORKERS = 32
_ROWS_PER_WORKER = _Q // _NUM_WORKERS
_LANES = 16


def _lut_body(table_ref, out_ref):
    i = lax.broadcasted_iota(jnp.int32, (_TABLE_ROWS, _LUT_LEN), 1)
    b = lax.broadcasted_iota(jnp.int32, (_TABLE_ROWS, _LUT_LEN), 0)
    max_exact = _NUM_BUCKETS // 2
    safe_rp = jnp.maximum(i.astype(jnp.float32), 1.0)
    rp_if_large = max_exact + (
        jnp.log(safe_rp / max_exact)
        / math.log(_MAX_DISTANCE / max_exact)
        * (_NUM_BUCKETS - max_exact)
    ).astype(jnp.int32)
    rp_if_large = jnp.minimum(rp_if_large, _NUM_BUCKETS - 1)
    abs_bucket = jnp.where(i < max_exact, i, rp_if_large)
    idx = jnp.where(i < _ABS_LEN, abs_bucket, i - _ABS_LEN + _NUM_BUCKETS)
    onehot = (idx == b).astype(jnp.float32)
    out_ref[...] = lax.dot_general(
        table_ref[...],
        onehot,
        (((0,), (0,)), ((), ())),
        preferred_element_type=jnp.float32,
        precision=lax.Precision.HIGHEST,
    )


def _build_lut(table):
    return pl.pallas_call(
        _lut_body,
        out_shape=jax.ShapeDtypeStruct((_NUM_HEADS, _LUT_LEN), jnp.float32),
    )(table)


def _sc_body(
    lut_hbm, qseg_hbm, kseg_hbm, out_hbm, lut_v, ks_v, qs_v, row_a, row_b,
    sem_a, sem_b,
):
    c = lax.axis_index("c")
    s = lax.axis_index("s")
    w = s * 2 + c
    base_q = w * _ROWS_PER_WORKER

    pltpu.sync_copy(lut_hbm, lut_v)
    pltpu.sync_copy(kseg_hbm, ks_v)
    pltpu.sync_copy(qseg_hbm.at[pl.ds(base_q, _ROWS_PER_WORKER)], qs_v)

    lane_iota = lax.iota(jnp.int32, _LANES)

    def compute_row(qq, qi, buf):
        qsv = plsc.load_gather(qs_v, [jnp.broadcast_to(qi, (_LANES,))])
        segbase = _ABS_LEN + qsv * _NUM_SEGMENTS
        dbase = qq - lane_iota

        @plsc.parallel_loop(0, _K, _LANES, unroll=4)
        def k_body(k0):
            kv = ks_v[pl.ds(k0, _LANES)]
            dcl = jnp.clip(dbase - k0, 0, _ABS_LEN - 1)
            idx = jnp.where(qsv == kv, dcl, segbase + kv)
            for h in range(_NUM_HEADS):
                val = plsc.load_gather(
                    lut_v.at[pl.ds(h * _LUT_LEN, _LUT_LEN)], [idx]
                )
                buf[h, 0, pl.ds(k0, _LANES)] = val

    def drain(buf, sem):
        pltpu.make_async_copy(out_hbm.at[:, pl.ds(0, 1), :], buf, sem).wait()

    def q_body(qi, carry):
        q0 = base_q + 2 * qi

        @pl.when(qi > 0)
        def _():
            drain(row_a, sem_a)

        compute_row(q0, 2 * qi, row_a)
        pltpu.async_copy(row_a, out_hbm.at[:, pl.ds(q0, 1), :], sem_a)

        @pl.when(qi > 0)
        def _():
            drain(row_b, sem_b)

        compute_row(q0 + 1, 2 * qi + 1, row_b)
        pltpu.async_copy(row_b, out_hbm.at[:, pl.ds(q0 + 1, 1), :], sem_b)
        return carry

    lax.fori_loop(0, _ROWS_PER_WORKER // 2, q_body, 0)
    drain(row_a, sem_a)
    drain(row_b, sem_b)


@jax.jit
def _run(qseg, kseg, table):
    lut = _build_lut(table).reshape(-1)
    mesh = plsc.VectorSubcoreMesh(core_axis_name="c", subcore_axis_name="s")
    out = pl.kernel(
        _sc_body,
        out_type=jax.ShapeDtypeStruct((_NUM_HEADS, _Q, _K), jnp.float32),
        mesh=mesh,
        compiler_params=pltpu.CompilerParams(needs_layout_passes=False),
        scratch_types=[
            pltpu.VMEM((_NUM_HEADS * _LUT_LEN,), jnp.float32),
            pltpu.VMEM((_K,), jnp.int32),
            pltpu.VMEM((_ROWS_PER_WORKER,), jnp.int32),
            pltpu.VMEM((_NUM_HEADS, 1, _K), jnp.float32),
            pltpu.VMEM((_NUM_HEADS, 1, _K), jnp.float32),
            pltpu.SemaphoreType.DMA,
            pltpu.SemaphoreType.DMA,
        ],
    )(lut, qseg, kseg)
    return out


def kernel(key_pos, query_pos, key_segment, query_segment, relative_attention_bias):
    del key_pos, query_pos
    qseg = query_segment.reshape(_Q)
    kseg = key_segment.reshape(_K)
    out = _run(qseg, kseg, relative_attention_bias)
    return out.reshape(1, _NUM_HEADS, _Q, _K)


# speedup vs baseline: 1.0528x; 1.0410x over previous
import functools
import math

import jax
import jax.numpy as jnp
from jax import lax
from jax.experimental import pallas as pl
from jax.experimental.pallas import tpu as pltpu
from jax.experimental.pallas import tpu_sc as plsc

_NUM_HEADS = 16
_NUM_SEGMENTS = 32
_NUM_BUCKETS = 32
_MAX_DISTANCE = 128
_Q = 2048
_K = 2048
_ABS_LEN = 2048
_SEG_LEN = _NUM_SEGMENTS * _NUM_SEGMENTS
_LUT_LEN = _ABS_LEN + _SEG_LEN
_TABLE_ROWS = _SEG_LEN + _NUM_BUCKETS

_NUM_WORKERS = 32
_ROWS_PER_WORKER = _Q // _NUM_WORKERS
_LANES = 16


def _lut_body(table_ref, out_ref):
    i = lax.broadcasted_iota(jnp.int32, (_NUM_BUCKETS, _ABS_LEN), 1)
    b = lax.broadcasted_iota(jnp.int32, (_NUM_BUCKETS, _ABS_LEN), 0)
    max_exact = _NUM_BUCKETS // 2
    safe_rp = jnp.maximum(i.astype(jnp.float32), 1.0)
    rp_if_large = max_exact + (
        jnp.log(safe_rp / max_exact)
        / math.log(_MAX_DISTANCE / max_exact)
        * (_NUM_BUCKETS - max_exact)
    ).astype(jnp.int32)
    rp_if_large = jnp.minimum(rp_if_large, _NUM_BUCKETS - 1)
    abs_bucket = jnp.where(i < max_exact, i, rp_if_large)
    onehot = (abs_bucket == b).astype(jnp.float32)
    out_ref[...] = lax.dot_general(
        table_ref[...],
        onehot,
        (((0,), (0,)), ((), ())),
        preferred_element_type=jnp.float32,
        precision=lax.Precision.HIGHEST,
    )


def _build_lut(table):
    abs_lut = pl.pallas_call(
        _lut_body,
        out_shape=jax.ShapeDtypeStruct((_NUM_HEADS, _ABS_LEN), jnp.float32),
    )(table[: _NUM_BUCKETS])
    seg_lut = table[_NUM_BUCKETS :].T
    return jnp.concatenate([abs_lut, seg_lut], axis=1)


def _sc_body(
    lut_hbm, qseg_hbm, kseg_hbm, out_hbm, lut_v, ks_v, qs_v, row_a, row_b,
    sem_a, sem_b,
):
    c = lax.axis_index("c")
    s = lax.axis_index("s")
    w = s * 2 + c
    base_q = w * _ROWS_PER_WORKER

    pltpu.sync_copy(lut_hbm, lut_v)
    pltpu.sync_copy(kseg_hbm, ks_v)
    pltpu.sync_copy(qseg_hbm.at[pl.ds(base_q, _ROWS_PER_WORKER)], qs_v)

    lane_iota = lax.iota(jnp.int32, _LANES)

    def compute_row(qq, qi, buf):
        qsv = plsc.load_gather(qs_v, [jnp.broadcast_to(qi, (_LANES,))])
        segbase = _ABS_LEN + qsv * _NUM_SEGMENTS
        dbase = qq - lane_iota

        @plsc.parallel_loop(0, _K, _LANES, unroll=4)
        def k_body(k0):
            kv = ks_v[pl.ds(k0, _LANES)]
            dcl = jnp.clip(dbase - k0, 0, _ABS_LEN - 1)
            idx = jnp.where(qsv == kv, dcl, segbase + kv)
            for h in range(_NUM_HEADS):
                val = plsc.load_gather(
                    lut_v.at[pl.ds(h * _LUT_LEN, _LUT_LEN)], [idx]
                )
                buf[h, 0, pl.ds(k0, _LANES)] = val

    def drain(buf, sem):
        pltpu.make_async_copy(out_hbm.at[:, pl.ds(0, 1), :], buf, sem).wait()

    def q_body(qi, carry):
        q0 = base_q + 2 * qi

        @pl.when(qi > 0)
        def _():
            drain(row_a, sem_a)

        compute_row(q0, 2 * qi, row_a)
        pltpu.async_copy(row_a, out_hbm.at[:, pl.ds(q0, 1), :], sem_a)

        @pl.when(qi > 0)
        def _():
            drain(row_b, sem_b)

        compute_row(q0 + 1, 2 * qi + 1, row_b)
        pltpu.async_copy(row_b, out_hbm.at[:, pl.ds(q0 + 1, 1), :], sem_b)
        return carry

    lax.fori_loop(0, _ROWS_PER_WORKER // 2, q_body, 0)
    drain(row_a, sem_a)
    drain(row_b, sem_b)


@jax.jit
def _run(qseg, kseg, table):
    lut = _build_lut(table).reshape(-1)
    mesh = plsc.VectorSubcoreMesh(core_axis_name="c", subcore_axis_name="s")
    out = pl.kernel(
        _sc_body,
        out_type=jax.ShapeDtypeStruct((_NUM_HEADS, _Q, _K), jnp.float32),
        mesh=mesh,
        compiler_params=pltpu.CompilerParams(needs_layout_passes=False),
        scratch_types=[
            pltpu.VMEM((_NUM_HEADS * _LUT_LEN,), jnp.float32),
            pltpu.VMEM((_K,), jnp.int32),
            pltpu.VMEM((_ROWS_PER_WORKER,), jnp.int32),
            pltpu.VMEM((_NUM_HEADS, 1, _K), jnp.float32),
            pltpu.VMEM((_NUM_HEADS, 1, _K), jnp.float32),
            pltpu.SemaphoreType.DMA,
            pltpu.SemaphoreType.DMA,
        ],
    )(lut, qseg, kseg)
    return out


def kernel(key_pos, query_pos, key_segment, query_segment, relative_attention_bias):
    del key_pos, query_pos
    qseg = query_segment.reshape(_Q)
    kseg = key_segment.reshape(_K)
    out = _run(qseg, kseg, relative_attention_bias)
    return out.reshape(1, _NUM_HEADS, _Q, _K)
